# baseline (speedup 1.0000x reference)
import functools

import jax
import jax.numpy as jnp
from jax.experimental import pallas as pl
from jax.experimental.pallas import tpu as pltpu

N_ROWS = 5000
D_IN = 12544
D_HID = 1024
BR = 200
NR = N_ROWS // BR
WSLAB = 448
NW = D_IN // WSLAB
NSTEPS = NW + NR
C1 = 4
C4 = 12
CH = 16


def _boxhead_body(x_ref, w1_ref, b1_ref, w2_ref, b2_ref, wh_ref, bh_ref,
                  cls_ref, box_ref, w1b_ref, h1_ref, acc0_ref):
    j = pl.program_id(0)

    @pl.when(j < NW)
    def _cast_w1():
        w1b_ref[pl.ds(j * WSLAB, WSLAB), :] = w1_ref[...].astype(jnp.bfloat16)

        @pl.when(j % 2 == 1)
        def _partial0():
            koff = (j // 2) * (2 * WSLAB)
            xb0 = x_ref[:, pl.ds(koff, 2 * WSLAB)].astype(jnp.bfloat16)
            part = jnp.dot(xb0, w1b_ref[pl.ds(koff, 2 * WSLAB), :],
                           preferred_element_type=jnp.float32)

            @pl.when(j == 1)
            def _():
                acc0_ref[...] = part

            @pl.when(j > 1)
            def _():
                acc0_ref[...] += part

            @pl.when(j == NW - 1)
            def _():
                h1_ref[...] = jnp.maximum(
                    acc0_ref[...] + b1_ref[...], 0.0).astype(jnp.bfloat16)

    @pl.when(j >= NW)
    def _steady():
        h2 = jnp.maximum(
            jnp.dot(h1_ref[...], w2_ref[...],
                    preferred_element_type=jnp.float32)
            + b2_ref[...], 0.0)
        heads = (jnp.dot(h2.astype(jnp.bfloat16), wh_ref[...],
                         preferred_element_type=jnp.float32) + bh_ref[...])
        cls_ref[...] = heads[:, :C1]
        box_ref[...] = heads[:, C1:]

        @pl.when(j < NSTEPS - 1)
        def _layer1():
            xb = x_ref[...].astype(jnp.bfloat16)
            pre = jnp.dot(xb, w1b_ref[...], preferred_element_type=jnp.float32)
            h1_ref[...] = jnp.maximum(
                pre + b1_ref[...], 0.0).astype(jnp.bfloat16)


def _clamp(lo, v, hi):
    return jnp.minimum(jnp.maximum(v, lo), hi)


@functools.partial(jax.jit, static_argnames=())
def kernel(feature_vectors, W1, b1, W2, b2, W3, b3, W4, b4):
    W2b = W2.astype(jnp.bfloat16)
    WHb = jnp.concatenate([W3, W4], axis=1).astype(jnp.bfloat16)
    bh = jnp.concatenate([b3, b4]).reshape(1, CH)
    out = pl.pallas_call(
        _boxhead_body,
        grid=(NSTEPS,),
        in_specs=[
            pl.BlockSpec((BR, D_IN),
                         lambda j: (_clamp(0, j - NW + 1, NR - 1), 0)),
            pl.BlockSpec((WSLAB, D_HID),
                         lambda j: (_clamp(0, j, NW - 1), 0)),
            pl.BlockSpec((1, D_HID), lambda j: (0, 0)),
            pl.BlockSpec((D_HID, D_HID), lambda j: (0, 0)),
            pl.BlockSpec((1, D_HID), lambda j: (0, 0)),
            pl.BlockSpec((D_HID, CH), lambda j: (0, 0)),
            pl.BlockSpec((1, CH), lambda j: (0, 0)),
        ],
        out_specs=[
            pl.BlockSpec((BR, C1), lambda j: (_clamp(0, j - NW, NR - 1), 0)),
            pl.BlockSpec((BR, C4), lambda j: (_clamp(0, j - NW, NR - 1), 0)),
        ],
        out_shape=[
            jax.ShapeDtypeStruct((N_ROWS, C1), jnp.float32),
            jax.ShapeDtypeStruct((N_ROWS, C4), jnp.float32),
        ],
        scratch_shapes=[
            pltpu.VMEM((D_IN, D_HID), jnp.bfloat16),
            pltpu.VMEM((BR, D_HID), jnp.bfloat16),
            pltpu.VMEM((BR, D_HID), jnp.float32),
        ],
        compiler_params=pltpu.CompilerParams(
            dimension_semantics=("arbitrary",),
        ),
    )(feature_vectors, W1, b1.reshape(1, -1), W2b, b2.reshape(1, -1),
      WHb, bh)
    return (out[0], out[1])


# speedup vs baseline: 1.1127x; 1.1127x over previous
import functools

import jax
import jax.numpy as jnp
from jax.experimental import pallas as pl
from jax.experimental.pallas import tpu as pltpu

N_ROWS = 5000
D_IN = 12544
D_HID = 1024
BR = 200
NR = N_ROWS // BR
WSLAB = 896
NW = D_IN // WSLAB
NSTEPS = NW + NR
C1 = 4
C4 = 12
CH = 16


def _boxhead_body(x_ref, w1_ref, b1_ref, w2_ref, b2_ref, wh_ref, bh_ref,
                  cls_ref, box_ref, w1b_ref):
    j = pl.program_id(0)

    @pl.when(j < NW)
    def _cast_w1():
        w1b_ref[pl.ds(j * WSLAB, WSLAB), :] = w1_ref[...].astype(jnp.bfloat16)

    @pl.when(j >= NW)
    def _stripe():
        xb = x_ref[...].astype(jnp.bfloat16)
        h1 = jnp.maximum(
            jnp.dot(xb, w1b_ref[...], preferred_element_type=jnp.float32)
            + b1_ref[...], 0.0)
        h2 = jnp.maximum(
            jnp.dot(h1.astype(jnp.bfloat16), w2_ref[...],
                    preferred_element_type=jnp.float32)
            + b2_ref[...], 0.0)
        heads = (jnp.dot(h2.astype(jnp.bfloat16), wh_ref[...],
                         preferred_element_type=jnp.float32) + bh_ref[...])
        cls_ref[...] = heads[:, :C1]
        box_ref[...] = heads[:, C1:]


def _clamp(lo, v, hi):
    return jnp.minimum(jnp.maximum(v, lo), hi)


@functools.partial(jax.jit, static_argnames=())
def kernel(feature_vectors, W1, b1, W2, b2, W3, b3, W4, b4):
    W2b = W2.astype(jnp.bfloat16)
    WHb = jnp.concatenate([W3, W4], axis=1).astype(jnp.bfloat16)
    bh = jnp.concatenate([b3, b4]).reshape(1, CH)
    out = pl.pallas_call(
        _boxhead_body,
        grid=(NSTEPS,),
        in_specs=[
            pl.BlockSpec((BR, D_IN),
                         lambda j: (_clamp(0, j - NW, NR - 1), 0)),
            pl.BlockSpec((WSLAB, D_HID),
                         lambda j: (_clamp(0, j, NW - 1), 0)),
            pl.BlockSpec((1, D_HID), lambda j: (0, 0)),
            pl.BlockSpec((D_HID, D_HID), lambda j: (0, 0)),
            pl.BlockSpec((1, D_HID), lambda j: (0, 0)),
            pl.BlockSpec((D_HID, CH), lambda j: (0, 0)),
            pl.BlockSpec((1, CH), lambda j: (0, 0)),
        ],
        out_specs=[
            pl.BlockSpec((BR, C1), lambda j: (_clamp(0, j - NW, NR - 1), 0)),
            pl.BlockSpec((BR, C4), lambda j: (_clamp(0, j - NW, NR - 1), 0)),
        ],
        out_shape=[
            jax.ShapeDtypeStruct((N_ROWS, C1), jnp.float32),
            jax.ShapeDtypeStruct((N_ROWS, C4), jnp.float32),
        ],
        scratch_shapes=[
            pltpu.VMEM((D_IN, D_HID), jnp.bfloat16),
        ],
        compiler_params=pltpu.CompilerParams(
            dimension_semantics=("arbitrary",),
        ),
    )(feature_vectors, W1, b1.reshape(1, -1), W2b, b2.reshape(1, -1),
      WHb, bh)
    return (out[0], out[1])


# speedup vs baseline: 1.1150x; 1.0021x over previous
import functools

import jax
import jax.numpy as jnp
from jax.experimental import pallas as pl
from jax.experimental.pallas import tpu as pltpu

N_ROWS = 5000
D_IN = 12544
D_HID = 1024
BR = 200
NR = N_ROWS // BR
WSLAB = 896
NW = D_IN // WSLAB
NSTEPS = NW + NR
C1 = 4
C4 = 12
CH = 16


def _boxhead_body(x_ref, w1_ref, b1_ref, w2_ref, b2_ref, wh_ref, bh_ref,
                  out_ref, w1b_ref):
    j = pl.program_id(0)

    @pl.when(j < NW)
    def _cast_w1():
        w1b_ref[pl.ds(j * WSLAB, WSLAB), :] = w1_ref[...].astype(jnp.bfloat16)

    @pl.when(j >= NW)
    def _stripe():
        xb = x_ref[...].astype(jnp.bfloat16)
        h1 = jnp.maximum(
            jnp.dot(xb, w1b_ref[...], preferred_element_type=jnp.float32)
            + b1_ref[...], 0.0)
        h2 = jnp.maximum(
            jnp.dot(h1.astype(jnp.bfloat16), w2_ref[...],
                    preferred_element_type=jnp.float32)
            + b2_ref[...], 0.0)
        out_ref[...] = (jnp.dot(h2.astype(jnp.bfloat16), wh_ref[...],
                                preferred_element_type=jnp.float32)
                        + bh_ref[...])


def _clamp(lo, v, hi):
    return jnp.minimum(jnp.maximum(v, lo), hi)


@functools.partial(jax.jit, static_argnames=())
def kernel(feature_vectors, W1, b1, W2, b2, W3, b3, W4, b4):
    W2b = W2.astype(jnp.bfloat16)
    WHb = jnp.concatenate([W3, W4], axis=1).astype(jnp.bfloat16)
    bh = jnp.concatenate([b3, b4]).reshape(1, CH)
    out = pl.pallas_call(
        _boxhead_body,
        grid=(NSTEPS,),
        in_specs=[
            pl.BlockSpec((BR, D_IN),
                         lambda j: (_clamp(0, j - NW, NR - 1), 0)),
            pl.BlockSpec((WSLAB, D_HID),
                         lambda j: (_clamp(0, j, NW - 1), 0)),
            pl.BlockSpec((1, D_HID), lambda j: (0, 0)),
            pl.BlockSpec((D_HID, D_HID), lambda j: (0, 0)),
            pl.BlockSpec((1, D_HID), lambda j: (0, 0)),
            pl.BlockSpec((D_HID, CH), lambda j: (0, 0)),
            pl.BlockSpec((1, CH), lambda j: (0, 0)),
        ],
        out_specs=pl.BlockSpec((BR, CH), lambda j: (_clamp(0, j - NW, NR - 1), 0)),
        out_shape=jax.ShapeDtypeStruct((N_ROWS, CH), jnp.float32),
        scratch_shapes=[
            pltpu.VMEM((D_IN, D_HID), jnp.bfloat16),
        ],
        compiler_params=pltpu.CompilerParams(
            dimension_semantics=("arbitrary",),
        ),
    )(feature_vectors, W1, b1.reshape(1, -1), W2b, b2.reshape(1, -1),
      WHb, bh)
    return (out[:, :C1], out[:, C1:])
